# baseline (speedup 1.0000x reference)
import functools

import jax
import jax.numpy as jnp
from jax import lax
from jax.experimental import pallas as pl
from jax.experimental.pallas import tpu as pltpu
from jax.experimental.pallas import tpu_sc as plsc

N_NODES = 10000
N_EDGES = 160000
D = 256
HD = 128
DIST_DIM = 16
HID = 768

NW = 32
CHUNK = 128
E_PAD = 163840
NSLAB = 16
EH = E_PAD // NSLAB
SPS = 16 // NSLAB

EPT = E_PAD // 16
SCHUNKS = EPT // CHUNK
ACC_ROWS = 10112
ZROWS = ACC_ROWS // 16

EDGE_BLK = 2048
NODE_BLK = 2000

_mesh = plsc.VectorSubcoreMesh(core_axis_name="c", subcore_axis_name="s")


GROWS = 2 * EH // CHUNK
GPW = GROWS // NW
GNB = 3


@functools.partial(
    pl.kernel,
    out_type=jax.ShapeDtypeStruct((2 * EH, D // 2), jnp.int32),
    mesh=_mesh,
    scratch_types=[
        pltpu.VMEM((GPW, CHUNK), jnp.int32),
        pltpu.VMEM((CHUNK, D // 2), jnp.int32),
        pltpu.VMEM((CHUNK, D // 2), jnp.int32),
        pltpu.VMEM((CHUNK, D // 2), jnp.int32),
        pltpu.SemaphoreType.DMA,
        pltpu.SemaphoreType.DMA,
        pltpu.SemaphoreType.DMA,
        pltpu.SemaphoreType.DMA,
        pltpu.SemaphoreType.DMA,
        pltpu.SemaphoreType.DMA,
    ],
)
def _gather_sc(xn_hbm, idx2_hbm, gout_hbm,
               idxs, rows0, rows1, rows2, g0, g1, g2, w0, w1, w2):
    c = lax.axis_index("c")
    s = lax.axis_index("s")
    wid = c * 16 + s
    cbase = wid * GPW
    rows = [rows0, rows1, rows2]
    semg = [g0, g1, g2]
    semw = [w0, w1, w2]

    pltpu.sync_copy(idx2_hbm.at[wid], idxs)

    def start_g(j, b):
        pltpu.async_copy(xn_hbm.at[idxs.at[j]], rows[b], semg[b])

    def wait_g(j, b):
        pltpu.make_async_copy(xn_hbm.at[idxs.at[j]], rows[b], semg[b]).wait()

    def out_ref(j):
        return gout_hbm.at[pl.ds((cbase + j) * CHUNK, CHUNK)]

    def start_w(j, b):
        pltpu.async_copy(rows[b], out_ref(j), semw[b])

    def wait_w(j, b):
        pltpu.make_async_copy(rows[b], out_ref(j), semw[b]).wait()

    def body(i, carry):
        for v in range(GNB):
            j = i * GNB + v
            b = v

            @pl.when((j >= GNB) & (j - GNB < GPW))
            def _():
                wait_w(j - GNB, b)

            @pl.when(j < GPW)
            def _():
                start_g(j, b)

            q = j - (GNB - 1)
            bq = (v + 1) % GNB

            @pl.when((q >= 0) & (q < GPW))
            def _():
                wait_g(q, bq)
                start_w(q, bq)

        return carry

    lax.fori_loop(0, (GPW + 2 * GNB - 1) // GNB, body, 0)


SNB = 2


@functools.partial(
    pl.kernel,
    out_type=(jax.ShapeDtypeStruct((ACC_ROWS, HD), jnp.float32),
              jax.ShapeDtypeStruct((ACC_ROWS, HD), jnp.float32)),
    mesh=_mesh,
    scratch_types=[
        pltpu.VMEM((SCHUNKS, CHUNK), jnp.int32),
        pltpu.VMEM((CHUNK, HD), jnp.float32),
        pltpu.VMEM((CHUNK, HD), jnp.float32),
        pltpu.VMEM_SHARED((ACC_ROWS, HD), jnp.float32),
        pltpu.SemaphoreType.DMA,
        pltpu.SemaphoreType.DMA,
        pltpu.SemaphoreType.DMA,
        pltpu.SemaphoreType.DMA,
    ],
)
def _scatter_sc(*args):
    slabs0 = list(args[:NSLAB])
    slabs1 = list(args[NSLAB:2 * NSLAB])
    (dst2_hbm, zeros_hbm, agg0_hbm, agg1_hbm,
     ids, r0, r1, acc, l0, l1, t0, t1) = args[2 * NSLAB:]
    c = lax.axis_index("c")
    s = lax.axis_index("s")
    rows = [r0, r1]
    seml = [l0, l1]
    sems = [t0, t1]

    pltpu.sync_copy(zeros_hbm, acc.at[pl.ds(s * ZROWS, ZROWS)])
    pltpu.sync_copy(dst2_hbm.at[pl.ds(s * SCHUNKS, SCHUNKS)], ids)
    plsc.subcore_barrier()

    def run(msg_hbm, base):
        def in_ref(j):
            return msg_hbm.at[pl.ds((base * SCHUNKS + j) * CHUNK, CHUNK)]

        def start_l(j, b):
            pltpu.async_copy(in_ref(j), rows[b], seml[b])

        def wait_l(j, b):
            pltpu.make_async_copy(in_ref(j), rows[b], seml[b]).wait()

        def start_s(j, b):
            pltpu.async_copy(rows[b], acc.at[ids.at[j]], sems[b], add=True)

        def wait_s(j, b):
            pltpu.make_async_copy(rows[b], acc.at[ids.at[j]], sems[b]).wait()

        def body(i, carry):
            for v in range(SNB):
                j = i * SNB + v
                b = v

                @pl.when((j >= SNB) & (j - SNB < SCHUNKS))
                def _():
                    wait_s(j - SNB, b)

                @pl.when(j < SCHUNKS)
                def _():
                    start_l(j, b)

                q = j - (SNB - 1)
                bq = (v + 1) % SNB

                @pl.when((q >= 0) & (q < SCHUNKS))
                def _():
                    wait_l(q, bq)
                    start_s(q, bq)

            return carry

        lax.fori_loop(0, (SCHUNKS + 2 * SNB - 1) // SNB, body, 0)

    for k in range(NSLAB):
        lo, hi = k * SPS, (k + 1) * SPS

        @pl.when((c == 0) & (s >= lo) & (s < hi))
        def _(k=k, lo=lo):
            run(slabs0[k], s - lo)

        @pl.when((c == 1) & (s >= lo) & (s < hi))
        def _(k=k, lo=lo):
            run(slabs1[k], s - lo)

    plsc.subcore_barrier()

    @pl.when(c == 0)
    def _():
        pltpu.sync_copy(acc.at[pl.ds(s * ZROWS, ZROWS)],
                        agg0_hbm.at[pl.ds(s * ZROWS, ZROWS)])

    @pl.when(c == 1)
    def _():
        pltpu.sync_copy(acc.at[pl.ds(s * ZROWS, ZROWS)],
                        agg1_hbm.at[pl.ds(s * ZROWS, ZROWS)])


def _ln1_body(x_ref, g_ref, b_ref, out_ref):
    v = x_ref[...]
    mu = jnp.mean(v, axis=-1, keepdims=True)
    var = jnp.mean((v - mu) ** 2, axis=-1, keepdims=True)
    y = (v - mu) * lax.rsqrt(var + 1e-5) * g_ref[...] + b_ref[...]
    lo = lax.bitcast_convert_type(y[:, :HD].astype(jnp.bfloat16), jnp.uint16)
    hi = lax.bitcast_convert_type(y[:, HD:].astype(jnp.bfloat16), jnp.uint16)
    word = lo.astype(jnp.uint32) | (hi.astype(jnp.uint32) << 16)
    out_ref[...] = lax.bitcast_convert_type(word, jnp.int32)


def _ln1_tc(x, g1, b1):
    full = lambda *shape: pl.BlockSpec(shape, lambda i: (0,) * len(shape))
    return pl.pallas_call(
        _ln1_body,
        grid=(N_NODES // NODE_BLK,),
        in_specs=[pl.BlockSpec((NODE_BLK, D), lambda i: (i, 0)),
                  full(D), full(D)],
        out_specs=pl.BlockSpec((NODE_BLK, HD), lambda i: (i, 0)),
        out_shape=jax.ShapeDtypeStruct((N_NODES, HD), jnp.int32),
    )(x, g1, b1)


def _unpack(g):
    w = lax.bitcast_convert_type(g, jnp.uint32)
    lo = lax.bitcast_convert_type((w & 0xFFFF).astype(jnp.uint16),
                                  jnp.bfloat16)
    hi = lax.bitcast_convert_type((w >> 16).astype(jnp.uint16),
                                  jnp.bfloat16)
    return lo, hi


def _edge_body(gd_ref, gs_ref, dist_ref,
               wa0_ref, wa1_ref, wb0_ref, wb1_ref,
               wc_ref, bm1_ref, wm2_ref, bm2_ref,
               msg0_ref, msg1_ref):
    xd0, xd1 = _unpack(gd_ref[...])
    xs0, xs1 = _unpack(gs_ref[...])
    dd = dist_ref[...].astype(jnp.bfloat16)
    h = (jnp.dot(xd0, wa0_ref[...], preferred_element_type=jnp.float32)
         + jnp.dot(xd1, wa1_ref[...], preferred_element_type=jnp.float32)
         + jnp.dot(xs0, wb0_ref[...], preferred_element_type=jnp.float32)
         + jnp.dot(xs1, wb1_ref[...], preferred_element_type=jnp.float32)
         + jnp.dot(dd, wc_ref[...], preferred_element_type=jnp.float32)
         + bm1_ref[...])
    h = jnp.where(h >= 0, h, 0.01 * h)
    msg = (jnp.dot(h.astype(jnp.bfloat16), wm2_ref[...],
                   preferred_element_type=jnp.float32)
           + bm2_ref[...])
    msg0_ref[...] = msg[:, :HD]
    msg1_ref[...] = msg[:, HD:]


def _edge_tc(gout, dist, wa0, wa1, wb0, wb1, wc, bm1, wm2, bm2):
    grid = EH // EDGE_BLK
    full = lambda *shape: pl.BlockSpec(shape, lambda i: (0,) * len(shape))
    return pl.pallas_call(
        _edge_body,
        grid=(grid,),
        in_specs=[
            pl.BlockSpec((EDGE_BLK, HD), lambda i: (i, 0)),
            pl.BlockSpec((EDGE_BLK, HD), lambda i: (i + EH // EDGE_BLK, 0)),
            pl.BlockSpec((EDGE_BLK, DIST_DIM), lambda i: (i, 0)),
            full(HD, HID), full(HD, HID), full(HD, HID), full(HD, HID),
            full(DIST_DIM, HID), full(HID),
            full(HID, D), full(D),
        ],
        out_specs=[pl.BlockSpec((EDGE_BLK, HD), lambda i: (i, 0)),
                   pl.BlockSpec((EDGE_BLK, HD), lambda i: (i, 0))],
        out_shape=(jax.ShapeDtypeStruct((EH, HD), jnp.float32),
                   jax.ShapeDtypeStruct((EH, HD), jnp.float32)),
    )(gout, gout, dist, wa0, wa1, wb0, wb1, wc, bm1, wm2, bm2)


def _ffn_body(x_ref, a0_ref, a1_ref, g2_ref, b2_ref, wf1_ref, bf1_ref,
              wf2_ref, bf2_ref, out_ref):
    agg = jnp.concatenate([a0_ref[...], a1_ref[...]], axis=-1)
    x2 = x_ref[...] + agg
    mu = jnp.mean(x2, axis=-1, keepdims=True)
    var = jnp.mean((x2 - mu) ** 2, axis=-1, keepdims=True)
    xn = (x2 - mu) * lax.rsqrt(var + 1e-5) * g2_ref[...] + b2_ref[...]
    h2 = jnp.dot(xn.astype(jnp.bfloat16), wf1_ref[...],
                 preferred_element_type=jnp.float32) + bf1_ref[...]
    h2 = jnp.where(h2 >= 0, h2, 0.01 * h2)
    out_ref[...] = x2 + jnp.dot(h2.astype(jnp.bfloat16), wf2_ref[...],
                                preferred_element_type=jnp.float32) + bf2_ref[...]


def _ffn_tc(x, agg0, agg1, g2, b2, wf1, bf1, wf2, bf2):
    full = lambda *shape: pl.BlockSpec(shape, lambda i: (0,) * len(shape))
    return pl.pallas_call(
        _ffn_body,
        grid=(N_NODES // NODE_BLK,),
        in_specs=[
            pl.BlockSpec((NODE_BLK, D), lambda i: (i, 0)),
            pl.BlockSpec((NODE_BLK, HD), lambda i: (i, 0)),
            pl.BlockSpec((NODE_BLK, HD), lambda i: (i, 0)),
            full(D), full(D), full(D, HID), full(HID), full(HID, D), full(D),
        ],
        out_specs=pl.BlockSpec((NODE_BLK, D), lambda i: (i, 0)),
        out_shape=jax.ShapeDtypeStruct((N_NODES, D), jnp.float32),
    )(x, agg0, agg1, g2, b2, wf1, bf1, wf2, bf2)


def kernel(x, edge_index, dist_embedding, gamma1, beta1, gamma2, beta2,
           Wm1, bm1, Wm2, bm2, Wf1, bf1, Wf2, bf2):
    src = edge_index[0].astype(jnp.int32)
    dst = edge_index[1].astype(jnp.int32)
    pad = E_PAD - N_EDGES
    src_g = jnp.pad(src, (0, pad))
    dst_g = jnp.pad(dst, (0, pad))
    idx2 = [jnp.concatenate([dst_g[k * EH:(k + 1) * EH],
                             src_g[k * EH:(k + 1) * EH]]).reshape(NW, GPW, CHUNK)
            for k in range(NSLAB)]
    dst2 = jnp.pad(dst, (0, pad),
                   constant_values=N_NODES).reshape(E_PAD // CHUNK, CHUNK)
    dist_p = jnp.pad(dist_embedding, ((0, pad), (0, 0)))
    zeros = jnp.zeros((ZROWS, HD), jnp.float32)

    bf = jnp.bfloat16
    wa0, wa1 = Wm1[:HD].astype(bf), Wm1[HD:D].astype(bf)
    wb0, wb1 = Wm1[D:D + HD].astype(bf), Wm1[D + HD:2 * D].astype(bf)
    wc = Wm1[2 * D:].astype(bf)
    wm2 = Wm2.astype(bf)

    xn32 = _ln1_tc(x, gamma1, beta1)
    gouts = [_gather_sc(xn32, idx2[k]) for k in range(NSLAB)]
    msgs = [_edge_tc(gouts[k], dist_p[k * EH:(k + 1) * EH],
                     wa0, wa1, wb0, wb1, wc, bm1, wm2, bm2)
            for k in range(NSLAB)]
    agg0, agg1 = _scatter_sc(*[m[0] for m in msgs], *[m[1] for m in msgs],
                             dst2, zeros)
    return _ffn_tc(x, agg0[:N_NODES], agg1[:N_NODES],
                   gamma2, beta2, Wf1.astype(bf), bf1, Wf2.astype(bf), bf2)


# speedup vs baseline: 1.0221x; 1.0221x over previous
import functools

import jax
import jax.numpy as jnp
from jax import lax
from jax.experimental import pallas as pl
from jax.experimental.pallas import tpu as pltpu
from jax.experimental.pallas import tpu_sc as plsc

N_NODES = 10000
N_EDGES = 160000
D = 256
HD = 128
DIST_DIM = 16
HID = 768

NW = 32
CHUNK = 128
E_PAD = 163840
NSLAB = 8
EH = E_PAD // NSLAB
SPS = 16 // NSLAB

EPT = E_PAD // 16
SCHUNKS = EPT // CHUNK
ACC_ROWS = 10112
ZROWS = ACC_ROWS // 16

EDGE_BLK = 2048
NODE_BLK = 2000

_mesh = plsc.VectorSubcoreMesh(core_axis_name="c", subcore_axis_name="s")


GROWS = 2 * EH // CHUNK
GPW = GROWS // NW
GNB = 4


@functools.partial(
    pl.kernel,
    out_type=jax.ShapeDtypeStruct((2 * EH, D // 2), jnp.int32),
    mesh=_mesh,
    scratch_types=[
        pltpu.VMEM((GPW, CHUNK), jnp.int32),
        pltpu.VMEM((CHUNK, D // 2), jnp.int32),
        pltpu.VMEM((CHUNK, D // 2), jnp.int32),
        pltpu.VMEM((CHUNK, D // 2), jnp.int32),
        pltpu.VMEM((CHUNK, D // 2), jnp.int32),
        pltpu.SemaphoreType.DMA,
        pltpu.SemaphoreType.DMA,
        pltpu.SemaphoreType.DMA,
        pltpu.SemaphoreType.DMA,
        pltpu.SemaphoreType.DMA,
        pltpu.SemaphoreType.DMA,
        pltpu.SemaphoreType.DMA,
        pltpu.SemaphoreType.DMA,
    ],
)
def _gather_sc(xn_hbm, idx2_hbm, gout_hbm,
               idxs, rows0, rows1, rows2, rows3,
               g0, g1, g2, g3, w0, w1, w2, w3):
    c = lax.axis_index("c")
    s = lax.axis_index("s")
    wid = c * 16 + s
    cbase = wid * GPW
    rows = [rows0, rows1, rows2, rows3]
    semg = [g0, g1, g2, g3]
    semw = [w0, w1, w2, w3]

    pltpu.sync_copy(idx2_hbm.at[wid], idxs)

    def start_g(j, b):
        pltpu.async_copy(xn_hbm.at[idxs.at[j]], rows[b], semg[b])

    def wait_g(j, b):
        pltpu.make_async_copy(xn_hbm.at[idxs.at[j]], rows[b], semg[b]).wait()

    def out_ref(j):
        return gout_hbm.at[pl.ds((cbase + j) * CHUNK, CHUNK)]

    def start_w(j, b):
        pltpu.async_copy(rows[b], out_ref(j), semw[b])

    def wait_w(j, b):
        pltpu.make_async_copy(rows[b], out_ref(j), semw[b]).wait()

    def body(i, carry):
        for v in range(GNB):
            j = i * GNB + v
            b = v

            @pl.when((j >= GNB) & (j - GNB < GPW))
            def _():
                wait_w(j - GNB, b)

            @pl.when(j < GPW)
            def _():
                start_g(j, b)

            q = j - (GNB - 1)
            bq = (v + 1) % GNB

            @pl.when((q >= 0) & (q < GPW))
            def _():
                wait_g(q, bq)
                start_w(q, bq)

        return carry

    lax.fori_loop(0, (GPW + 2 * GNB - 1) // GNB, body, 0)


SNB = 2


@functools.partial(
    pl.kernel,
    out_type=(jax.ShapeDtypeStruct((ACC_ROWS, HD), jnp.float32),
              jax.ShapeDtypeStruct((ACC_ROWS, HD), jnp.float32)),
    mesh=_mesh,
    scratch_types=[
        pltpu.VMEM((SCHUNKS, CHUNK), jnp.int32),
        pltpu.VMEM((CHUNK, HD), jnp.float32),
        pltpu.VMEM((CHUNK, HD), jnp.float32),
        pltpu.VMEM_SHARED((ACC_ROWS, HD), jnp.float32),
        pltpu.SemaphoreType.DMA,
        pltpu.SemaphoreType.DMA,
        pltpu.SemaphoreType.DMA,
        pltpu.SemaphoreType.DMA,
    ],
)
def _scatter_sc(*args):
    slabs0 = list(args[:NSLAB])
    slabs1 = list(args[NSLAB:2 * NSLAB])
    (dst2_hbm, zeros_hbm, agg0_hbm, agg1_hbm,
     ids, r0, r1, acc, l0, l1, t0, t1) = args[2 * NSLAB:]
    c = lax.axis_index("c")
    s = lax.axis_index("s")
    rows = [r0, r1]
    seml = [l0, l1]
    sems = [t0, t1]

    pltpu.sync_copy(zeros_hbm, acc.at[pl.ds(s * ZROWS, ZROWS)])
    pltpu.sync_copy(dst2_hbm.at[pl.ds(s * SCHUNKS, SCHUNKS)], ids)
    plsc.subcore_barrier()

    def run(msg_hbm, base):
        def in_ref(j):
            return msg_hbm.at[pl.ds((base * SCHUNKS + j) * CHUNK, CHUNK)]

        def start_l(j, b):
            pltpu.async_copy(in_ref(j), rows[b], seml[b])

        def wait_l(j, b):
            pltpu.make_async_copy(in_ref(j), rows[b], seml[b]).wait()

        def start_s(j, b):
            pltpu.async_copy(rows[b], acc.at[ids.at[j]], sems[b], add=True)

        def wait_s(j, b):
            pltpu.make_async_copy(rows[b], acc.at[ids.at[j]], sems[b]).wait()

        def body(i, carry):
            for v in range(SNB):
                j = i * SNB + v
                b = v

                @pl.when((j >= SNB) & (j - SNB < SCHUNKS))
                def _():
                    wait_s(j - SNB, b)

                @pl.when(j < SCHUNKS)
                def _():
                    start_l(j, b)

                q = j - (SNB - 1)
                bq = (v + 1) % SNB

                @pl.when((q >= 0) & (q < SCHUNKS))
                def _():
                    wait_l(q, bq)
                    start_s(q, bq)

            return carry

        lax.fori_loop(0, (SCHUNKS + 2 * SNB - 1) // SNB, body, 0)

    for k in range(NSLAB):
        lo, hi = k * SPS, (k + 1) * SPS

        @pl.when((c == 0) & (s >= lo) & (s < hi))
        def _(k=k, lo=lo):
            run(slabs0[k], s - lo)

        @pl.when((c == 1) & (s >= lo) & (s < hi))
        def _(k=k, lo=lo):
            run(slabs1[k], s - lo)

    plsc.subcore_barrier()

    @pl.when(c == 0)
    def _():
        pltpu.sync_copy(acc.at[pl.ds(s * ZROWS, ZROWS)],
                        agg0_hbm.at[pl.ds(s * ZROWS, ZROWS)])

    @pl.when(c == 1)
    def _():
        pltpu.sync_copy(acc.at[pl.ds(s * ZROWS, ZROWS)],
                        agg1_hbm.at[pl.ds(s * ZROWS, ZROWS)])


def _ln1_body(x_ref, g_ref, b_ref, out_ref):
    v = x_ref[...]
    mu = jnp.mean(v, axis=-1, keepdims=True)
    var = jnp.mean((v - mu) ** 2, axis=-1, keepdims=True)
    y = (v - mu) * lax.rsqrt(var + 1e-5) * g_ref[...] + b_ref[...]
    lo = lax.bitcast_convert_type(y[:, :HD].astype(jnp.bfloat16), jnp.uint16)
    hi = lax.bitcast_convert_type(y[:, HD:].astype(jnp.bfloat16), jnp.uint16)
    word = lo.astype(jnp.uint32) | (hi.astype(jnp.uint32) << 16)
    out_ref[...] = lax.bitcast_convert_type(word, jnp.int32)


def _ln1_tc(x, g1, b1):
    full = lambda *shape: pl.BlockSpec(shape, lambda i: (0,) * len(shape))
    return pl.pallas_call(
        _ln1_body,
        grid=(N_NODES // NODE_BLK,),
        in_specs=[pl.BlockSpec((NODE_BLK, D), lambda i: (i, 0)),
                  full(D), full(D)],
        out_specs=pl.BlockSpec((NODE_BLK, HD), lambda i: (i, 0)),
        out_shape=jax.ShapeDtypeStruct((N_NODES, HD), jnp.int32),
    )(x, g1, b1)


def _unpack(g):
    w = lax.bitcast_convert_type(g, jnp.uint32)
    lo = lax.bitcast_convert_type((w & 0xFFFF).astype(jnp.uint16),
                                  jnp.bfloat16)
    hi = lax.bitcast_convert_type((w >> 16).astype(jnp.uint16),
                                  jnp.bfloat16)
    return lo, hi


def _edge_body(gd_ref, gs_ref, dist_ref,
               wa0_ref, wa1_ref, wb0_ref, wb1_ref,
               wc_ref, bm1_ref, wm2_ref, bm2_ref,
               msg0_ref, msg1_ref):
    xd0, xd1 = _unpack(gd_ref[...])
    xs0, xs1 = _unpack(gs_ref[...])
    dd = dist_ref[...].astype(jnp.bfloat16)
    h = (jnp.dot(xd0, wa0_ref[...], preferred_element_type=jnp.float32)
         + jnp.dot(xd1, wa1_ref[...], preferred_element_type=jnp.float32)
         + jnp.dot(xs0, wb0_ref[...], preferred_element_type=jnp.float32)
         + jnp.dot(xs1, wb1_ref[...], preferred_element_type=jnp.float32)
         + jnp.dot(dd, wc_ref[...], preferred_element_type=jnp.float32)
         + bm1_ref[...])
    h = jnp.where(h >= 0, h, 0.01 * h)
    msg = (jnp.dot(h.astype(jnp.bfloat16), wm2_ref[...],
                   preferred_element_type=jnp.float32)
           + bm2_ref[...])
    msg0_ref[...] = msg[:, :HD]
    msg1_ref[...] = msg[:, HD:]


def _edge_tc(gout, dist, wa0, wa1, wb0, wb1, wc, bm1, wm2, bm2):
    grid = EH // EDGE_BLK
    full = lambda *shape: pl.BlockSpec(shape, lambda i: (0,) * len(shape))
    return pl.pallas_call(
        _edge_body,
        grid=(grid,),
        in_specs=[
            pl.BlockSpec((EDGE_BLK, HD), lambda i: (i, 0)),
            pl.BlockSpec((EDGE_BLK, HD), lambda i: (i + EH // EDGE_BLK, 0)),
            pl.BlockSpec((EDGE_BLK, DIST_DIM), lambda i: (i, 0)),
            full(HD, HID), full(HD, HID), full(HD, HID), full(HD, HID),
            full(DIST_DIM, HID), full(HID),
            full(HID, D), full(D),
        ],
        out_specs=[pl.BlockSpec((EDGE_BLK, HD), lambda i: (i, 0)),
                   pl.BlockSpec((EDGE_BLK, HD), lambda i: (i, 0))],
        out_shape=(jax.ShapeDtypeStruct((EH, HD), jnp.float32),
                   jax.ShapeDtypeStruct((EH, HD), jnp.float32)),
    )(gout, gout, dist, wa0, wa1, wb0, wb1, wc, bm1, wm2, bm2)


def _ffn_body(x_ref, a0_ref, a1_ref, g2_ref, b2_ref, wf1_ref, bf1_ref,
              wf2_ref, bf2_ref, out_ref):
    agg = jnp.concatenate([a0_ref[...], a1_ref[...]], axis=-1)
    x2 = x_ref[...] + agg
    mu = jnp.mean(x2, axis=-1, keepdims=True)
    var = jnp.mean((x2 - mu) ** 2, axis=-1, keepdims=True)
    xn = (x2 - mu) * lax.rsqrt(var + 1e-5) * g2_ref[...] + b2_ref[...]
    h2 = jnp.dot(xn.astype(jnp.bfloat16), wf1_ref[...],
                 preferred_element_type=jnp.float32) + bf1_ref[...]
    h2 = jnp.where(h2 >= 0, h2, 0.01 * h2)
    out_ref[...] = x2 + jnp.dot(h2.astype(jnp.bfloat16), wf2_ref[...],
                                preferred_element_type=jnp.float32) + bf2_ref[...]


def _ffn_tc(x, agg0, agg1, g2, b2, wf1, bf1, wf2, bf2):
    full = lambda *shape: pl.BlockSpec(shape, lambda i: (0,) * len(shape))
    return pl.pallas_call(
        _ffn_body,
        grid=(N_NODES // NODE_BLK,),
        in_specs=[
            pl.BlockSpec((NODE_BLK, D), lambda i: (i, 0)),
            pl.BlockSpec((NODE_BLK, HD), lambda i: (i, 0)),
            pl.BlockSpec((NODE_BLK, HD), lambda i: (i, 0)),
            full(D), full(D), full(D, HID), full(HID), full(HID, D), full(D),
        ],
        out_specs=pl.BlockSpec((NODE_BLK, D), lambda i: (i, 0)),
        out_shape=jax.ShapeDtypeStruct((N_NODES, D), jnp.float32),
    )(x, agg0, agg1, g2, b2, wf1, bf1, wf2, bf2)


def kernel(x, edge_index, dist_embedding, gamma1, beta1, gamma2, beta2,
           Wm1, bm1, Wm2, bm2, Wf1, bf1, Wf2, bf2):
    src = edge_index[0].astype(jnp.int32)
    dst = edge_index[1].astype(jnp.int32)
    pad = E_PAD - N_EDGES
    src_g = jnp.pad(src, (0, pad))
    dst_g = jnp.pad(dst, (0, pad))
    idx2 = [jnp.concatenate([dst_g[k * EH:(k + 1) * EH],
                             src_g[k * EH:(k + 1) * EH]]).reshape(NW, GPW, CHUNK)
            for k in range(NSLAB)]
    dst2 = jnp.pad(dst, (0, pad),
                   constant_values=N_NODES).reshape(E_PAD // CHUNK, CHUNK)
    dist_p = jnp.pad(dist_embedding, ((0, pad), (0, 0)))
    zeros = jnp.zeros((ZROWS, HD), jnp.float32)

    bf = jnp.bfloat16
    wa0, wa1 = Wm1[:HD].astype(bf), Wm1[HD:D].astype(bf)
    wb0, wb1 = Wm1[D:D + HD].astype(bf), Wm1[D + HD:2 * D].astype(bf)
    wc = Wm1[2 * D:].astype(bf)
    wm2 = Wm2.astype(bf)

    xn32 = _ln1_tc(x, gamma1, beta1)
    gouts = [_gather_sc(xn32, idx2[k]) for k in range(NSLAB)]
    msgs = [_edge_tc(gouts[k], dist_p[k * EH:(k + 1) * EH],
                     wa0, wa1, wb0, wb1, wc, bm1, wm2, bm2)
            for k in range(NSLAB)]
    agg0, agg1 = _scatter_sc(*[m[0] for m in msgs], *[m[1] for m in msgs],
                             dst2, zeros)
    return _ffn_tc(x, agg0[:N_NODES], agg1[:N_NODES],
                   gamma2, beta2, Wf1.astype(bf), bf1, Wf2.astype(bf), bf2)
